# speedup vs baseline: 3.3538x; 3.3538x over previous
import functools

import jax
import jax.numpy as jnp
from jax import lax
from jax.experimental import pallas as pl
from jax.experimental.pallas import tpu as pltpu
from jax.experimental.pallas import tpu_sc as plsc

EMBED_DIM = 128
NUM_CORES = 2
NUM_SUBCORES = 16
NUM_WORKERS = NUM_CORES * NUM_SUBCORES
CHUNK = 128
NBUF = 5


def _make_emb_kernel(n_total: int, vocab: int, d: int):
  per_w = n_total // NUM_WORKERS
  n_chunk = per_w // CHUNK
  n_outer = n_chunk // NBUF
  mesh = plsc.VectorSubcoreMesh(core_axis_name="c", subcore_axis_name="s")

  @functools.partial(
      pl.kernel,
      mesh=mesh,
      out_type=jax.ShapeDtypeStruct((n_total, d), jnp.float32),
      scratch_types=[
          pltpu.VMEM((n_chunk, CHUNK), jnp.int32),
          pltpu.VMEM((NBUF, CHUNK, d), jnp.float32),
      ] + [pltpu.SemaphoreType.DMA] * NBUF,
  )
  def emb(idx_hbm, tab_hbm, out_hbm, idx_v, rows_v, *gsems):
    wid = lax.axis_index("s") * NUM_CORES + lax.axis_index("c")
    base = wid * per_w
    pltpu.sync_copy(idx_hbm.at[wid], idx_v)

    def gather_start(chunk_i, b):
      return pltpu.async_copy(
          tab_hbm.at[idx_v.at[chunk_i]], rows_v.at[b], gsems[b])

    def gather_wait(chunk_i, b):
      pltpu.make_async_copy(
          tab_hbm.at[idx_v.at[chunk_i]], rows_v.at[b], gsems[b]).wait()

    def store(chunk_i, b):
      pltpu.sync_copy(
          rows_v.at[b], out_hbm.at[pl.ds(base + chunk_i * CHUNK, CHUNK)])

    for b in range(NBUF):
      gather_start(b, b)

    def outer(g):
      for b in range(NBUF):
        c = g * NBUF + b
        gather_wait(c, b)
        store(c, b)
        gather_start(c + NBUF, b)

    pl.loop(0, n_outer - 1)(outer)

    for b in range(NBUF):
      c = (n_outer - 1) * NBUF + b
      gather_wait(c, b)
      store(c, b)

  return emb


def kernel(word_indices, embedding_table):
  batch, seq = word_indices.shape
  vocab, d = embedding_table.shape
  n_total = batch * seq
  idx = word_indices.reshape(-1).astype(jnp.int32)
  idx3 = idx.reshape(NUM_WORKERS, n_total // (NUM_WORKERS * CHUNK), CHUNK)
  emb = _make_emb_kernel(n_total, vocab, d)
  out_flat = emb(idx3, embedding_table)
  return out_flat.reshape(batch, seq, d)
